# baseline (speedup 1.0000x reference)
import functools

import jax
import jax.numpy as jnp
from jax import lax
from jax.experimental import pallas as pl
from jax.experimental.pallas import tpu as pltpu, tpu_sc as plsc

_N_ATOMS = 1000000
_N_GROUND = 500000
_BODY_LEN = 4
_HEAD_LEN = 2
_LANES = 16

_TOTAL_G = _N_GROUND // _LANES
_NW = 32
_GPW = 980
_CG = 140
_NCH = _GPW // _CG
_BC = _CG * _LANES * _BODY_LEN
_HC = _CG * _LANES * _HEAD_LEN


def _make_sc_kernel():
    mesh = plsc.VectorSubcoreMesh(core_axis_name="c", subcore_axis_name="s")

    @functools.partial(
        pl.kernel,
        mesh=mesh,
        out_type=jax.ShapeDtypeStruct((3, _NW, _LANES), jnp.float32),
        compiler_params=pltpu.CompilerParams(needs_layout_passes=False),
        scratch_types=[
            pltpu.VMEM_SHARED((_N_ATOMS,), jnp.float32),
            pltpu.VMEM((_BC,), jnp.int32),
            pltpu.VMEM((_BC,), jnp.float32),
            pltpu.VMEM((_HC,), jnp.int32),
            pltpu.VMEM((_HC,), jnp.float32),
            pltpu.VMEM((_LANES,), jnp.float32),
            pltpu.SemaphoreType.DMA,
            pltpu.SemaphoreType.DMA,
        ],
    )
    def sc_kernel(pred_hbm, ain0, aout0, ain1, aout1, ain2, aout2, out_hbm,
                  spmem, bidx, bval, hidx, hval, stage, sem_b, sem_h):
        cid = lax.axis_index("c")
        sid = lax.axis_index("s")
        wid = sid * 2 + cid

        @pl.when(sid == 0)
        def _stage_table():
            pltpu.sync_copy(pred_hbm, spmem)

        plsc.subcore_barrier()

        base_g = jnp.minimum(wid * _GPW, _TOTAL_G - _GPW)
        skip = wid * _GPW - base_g

        for r, (ain, aout) in enumerate(
                ((ain0, aout0), (ain1, aout1), (ain2, aout2))):
            def chunk_body(c, acc, ain=ain, aout=aout):
                g0 = base_g + c * _CG
                pltpu.sync_copy(
                    ain.at[pl.ds(g0 * (_LANES * _BODY_LEN), _BC)], bidx)
                pltpu.sync_copy(
                    aout.at[pl.ds(g0 * (_LANES * _HEAD_LEN), _HC)], hidx)
                cp_b = pltpu.async_copy(spmem.at[bidx], bval, sem_b)
                cp_h = pltpu.async_copy(spmem.at[hidx], hval, sem_h)
                cp_b.wait()
                cp_h.wait()

                def group_body(g, a):
                    i16 = lax.iota(jnp.int32, _LANES)
                    b0 = g * (_LANES * _BODY_LEN)
                    h0 = g * (_LANES * _HEAD_LEN)
                    t = plsc.load_gather(bval, [i16 * _BODY_LEN + b0])
                    for j in range(1, _BODY_LEN):
                        t = t * plsc.load_gather(
                            bval, [i16 * _BODY_LEN + (b0 + j)])
                    for j in range(_HEAD_LEN):
                        t = t * (jnp.float32(1.0) - plsc.load_gather(
                            hval, [i16 * _HEAD_LEN + (h0 + j)]))
                    n = c * _CG + g
                    f = jnp.where(n >= skip, jnp.float32(1.0),
                                  jnp.float32(0.0))
                    return a + t * f

                return lax.fori_loop(0, _CG, group_body, acc)

            acc = lax.fori_loop(0, _NCH, chunk_body,
                                jnp.zeros((_LANES,), jnp.float32))
            stage[...] = acc
            pltpu.sync_copy(stage, out_hbm.at[r, wid])

    return sc_kernel


_SC_KERNEL = _make_sc_kernel()


def kernel(predictions, rule_weights, A_in_0, A_out_0, A_in_1, A_out_1,
           A_in_2, A_out_2):
    ains = [a.astype(jnp.float32).reshape(-1).astype(jnp.int32)
            for a in (A_in_0, A_in_1, A_in_2)]
    aouts = [a.astype(jnp.float32).reshape(-1).astype(jnp.int32)
             for a in (A_out_0, A_out_1, A_out_2)]
    partials = _SC_KERNEL(predictions, ains[0], aouts[0], ains[1], aouts[1],
                          ains[2], aouts[2])
    s = partials.sum(axis=(1, 2))
    return jnp.sum(rule_weights * s) / jnp.float32(_N_GROUND)


# speedup vs baseline: 11.3802x; 11.3802x over previous
import functools

import jax
import jax.numpy as jnp
from jax import lax
from jax.experimental import pallas as pl
from jax.experimental.pallas import tpu as pltpu, tpu_sc as plsc

_N_ATOMS = 1000000
_N_GROUND = 500000
_BODY_LEN = 4
_HEAD_LEN = 2
_LANES = 16

_BLK = 128
_NBLK = _N_GROUND // _BLK
_NW = 32
_BPW = 123
_CB = 41
_NCH = _BPW // _CB


def _make_sc_kernel():
    mesh = plsc.VectorSubcoreMesh(core_axis_name="c", subcore_axis_name="s")

    @functools.partial(
        pl.kernel,
        mesh=mesh,
        out_type=jax.ShapeDtypeStruct((3, _NW, _LANES), jnp.float32),
        compiler_params=pltpu.CompilerParams(
            needs_layout_passes=False,
            use_tc_tiling_on_sc=False,
        ),
        scratch_types=[
            pltpu.VMEM_SHARED((_N_ATOMS,), jnp.float32),
            pltpu.VMEM((_CB, _BODY_LEN, _BLK), jnp.int32),
            pltpu.VMEM((_CB, _BODY_LEN, _BLK), jnp.float32),
            pltpu.VMEM((_CB, _HEAD_LEN, _BLK), jnp.int32),
            pltpu.VMEM((_CB, _HEAD_LEN, _BLK), jnp.float32),
            pltpu.VMEM((_LANES,), jnp.float32),
            pltpu.SemaphoreType.DMA,
            pltpu.SemaphoreType.DMA,
        ],
    )
    def sc_kernel(pred_hbm, ain0, aout0, ain1, aout1, ain2, aout2, out_hbm,
                  spmem, bidx, bval, hidx, hval, stage, sem_b, sem_h):
        cid = lax.axis_index("c")
        sid = lax.axis_index("s")
        wid = sid * 2 + cid

        @pl.when(sid == 0)
        def _stage_table():
            pltpu.sync_copy(pred_hbm, spmem)

        plsc.subcore_barrier()

        base_b = jnp.minimum(wid * _BPW, _NBLK - _BPW)
        skip = wid * _BPW - base_b

        for r, (ain, aout) in enumerate(
                ((ain0, aout0), (ain1, aout1), (ain2, aout2))):
            def chunk_body(c, acc, ain=ain, aout=aout):
                b0 = base_b + c * _CB
                pltpu.sync_copy(ain.at[pl.ds(b0, _CB), :, :], bidx)
                pltpu.sync_copy(aout.at[pl.ds(b0, _CB), :, :], hidx)

                def fire(bl, _):
                    for j in range(_BODY_LEN):
                        pltpu.async_copy(spmem.at[bidx.at[bl, j]],
                                         bval.at[bl, j], sem_b)
                    for j in range(_HEAD_LEN):
                        pltpu.async_copy(spmem.at[hidx.at[bl, j]],
                                         hval.at[bl, j], sem_h)
                    return 0

                def drain(bl, _):
                    for j in range(_BODY_LEN):
                        pltpu.make_async_copy(spmem.at[bidx.at[bl, j]],
                                              bval.at[bl, j], sem_b).wait()
                    for j in range(_HEAD_LEN):
                        pltpu.make_async_copy(spmem.at[hidx.at[bl, j]],
                                              hval.at[bl, j], sem_h).wait()
                    return 0

                lax.fori_loop(0, _CB, fire, 0)
                lax.fori_loop(0, _CB, drain, 0)

                def block_body(bl, a):
                    n = c * _CB + bl
                    f = jnp.where(n >= skip, jnp.float32(1.0),
                                  jnp.float32(0.0))
                    for k in range(_BLK // _LANES):
                        s = k * _LANES
                        t = bval[bl, 0, pl.ds(s, _LANES)]
                        for j in range(1, _BODY_LEN):
                            t = t * bval[bl, j, pl.ds(s, _LANES)]
                        for j in range(_HEAD_LEN):
                            t = t * (jnp.float32(1.0)
                                     - hval[bl, j, pl.ds(s, _LANES)])
                        a = a + t * f
                    return a

                return lax.fori_loop(0, _CB, block_body, acc)

            acc = lax.fori_loop(0, _NCH, chunk_body,
                                jnp.zeros((_LANES,), jnp.float32))
            stage[...] = acc
            pltpu.sync_copy(stage, out_hbm.at[r, wid])

    return sc_kernel


_SC_KERNEL = _make_sc_kernel()


def _to_blocks(x, k):
    return x[:_NBLK * _BLK].reshape(_NBLK, _BLK, k).transpose(0, 2, 1)


def kernel(predictions, rule_weights, A_in_0, A_out_0, A_in_1, A_out_1,
           A_in_2, A_out_2):
    ains = [_to_blocks(a, _BODY_LEN) for a in (A_in_0, A_in_1, A_in_2)]
    aouts = [_to_blocks(a, _HEAD_LEN) for a in (A_out_0, A_out_1, A_out_2)]
    partials = _SC_KERNEL(predictions, ains[0], aouts[0], ains[1], aouts[1],
                          ains[2], aouts[2])
    s = partials.sum(axis=(1, 2))

    tail = []
    for a_in, a_out in ((A_in_0, A_out_0), (A_in_1, A_out_1),
                        (A_in_2, A_out_2)):
        tb = jnp.prod(jnp.take(predictions, a_in[_NBLK * _BLK:], axis=0),
                      axis=-1)
        th = jnp.prod(1.0 - jnp.take(predictions, a_out[_NBLK * _BLK:],
                                     axis=0), axis=-1)
        tail.append(jnp.sum(tb * th))
    s = s + jnp.stack(tail)

    return jnp.sum(rule_weights * s) / jnp.float32(_N_GROUND)


# speedup vs baseline: 12.4193x; 1.0913x over previous
import functools

import jax
import jax.numpy as jnp
from jax import lax
from jax.experimental import pallas as pl
from jax.experimental.pallas import tpu as pltpu, tpu_sc as plsc

_N_ATOMS = 1000000
_N_GROUND = 500000
_BODY_LEN = 4
_HEAD_LEN = 2
_LANES = 16

_BLK = 128
_NBLK = 3904
_NQ = _NBLK // 4
_NW = 32
_CBL = 16
_NCH = 8
_NRULE = 3


def _make_sc_kernel():
    mesh = plsc.VectorSubcoreMesh(core_axis_name="c", subcore_axis_name="s")

    @functools.partial(
        pl.kernel,
        mesh=mesh,
        out_type=jax.ShapeDtypeStruct((_NRULE, _NW, _LANES), jnp.float32),
        compiler_params=pltpu.CompilerParams(
            needs_layout_passes=False,
            use_tc_tiling_on_sc=False,
        ),
        scratch_types=[
            pltpu.VMEM_SHARED((_N_ATOMS,), jnp.float32),
            pltpu.VMEM((_CBL * _BODY_LEN, _BLK), jnp.int32),
            pltpu.VMEM((_CBL * _BODY_LEN, _BLK), jnp.float32),
            pltpu.VMEM((_CBL * _HEAD_LEN, _BLK), jnp.int32),
            pltpu.VMEM((_CBL * _HEAD_LEN, _BLK), jnp.float32),
            pltpu.VMEM((_CBL * _BODY_LEN, _BLK), jnp.int32),
            pltpu.VMEM((_CBL * _BODY_LEN, _BLK), jnp.float32),
            pltpu.VMEM((_CBL * _HEAD_LEN, _BLK), jnp.int32),
            pltpu.VMEM((_CBL * _HEAD_LEN, _BLK), jnp.float32),
            pltpu.VMEM((_LANES,), jnp.float32),
            pltpu.SemaphoreType.DMA, pltpu.SemaphoreType.DMA,
            pltpu.SemaphoreType.DMA, pltpu.SemaphoreType.DMA,
            pltpu.SemaphoreType.DMA, pltpu.SemaphoreType.DMA,
            pltpu.SemaphoreType.DMA, pltpu.SemaphoreType.DMA,
        ],
    )
    def sc_kernel(pred_hbm, ain0, aout0, ain1, aout1, ain2, aout2, out_hbm,
                  spmem, bidx0, bval0, hidx0, hval0,
                  bidx1, bval1, hidx1, hval1, stage,
                  ssb0, ssh0, ssb1, ssh1, sgb0, sgh0, sgb1, sgh1):
        cid = lax.axis_index("c")
        sid = lax.axis_index("s")
        wid = sid * 2 + cid

        @pl.when(sid == 0)
        def _stage_table():
            pltpu.sync_copy(pred_hbm, spmem)

        plsc.subcore_barrier()

        tq0 = jnp.where(wid < 16, 31 * wid, 496 + 30 * (wid - 16))
        nq = jnp.where(wid < 16, 31, 30)
        tq1 = tq0 + nq
        base_q = jnp.minimum(tq0, _NQ - _NCH * (_CBL // 4))

        rules = ((ain0, aout0), (ain1, aout1), (ain2, aout2))
        bufs = ((bidx0, bval0, hidx0, hval0, ssb0, ssh0, sgb0, sgh0),
                (bidx1, bval1, hidx1, hval1, ssb1, ssh1, sgb1, sgh1))
        sched = [(r, c) for r in range(_NRULE) for c in range(_NCH)]
        nsched = len(sched)

        def stage_i(i):
            r, c = sched[i]
            bidx, _, hidx, _, ssb, ssh, _, _ = bufs[i % 2]
            ain, aout = rules[r]
            q0 = base_q + c * (_CBL // 4)
            hs = pltpu.async_copy(
                ain.at[pl.ds(q0 * 4 * _BODY_LEN, _CBL * _BODY_LEN), :],
                bidx, ssb)
            hh = pltpu.async_copy(
                aout.at[pl.ds(q0 * 4 * _HEAD_LEN, _CBL * _HEAD_LEN), :],
                hidx, ssh)
            return hs, hh

        def fire_i(i, handles):
            hs, hh = handles
            hs.wait()
            hh.wait()
            bidx, bval, hidx, hval, _, _, sgb, sgh = bufs[i % 2]

            def fire(bl, _):
                for j in range(_BODY_LEN):
                    pltpu.async_copy(spmem.at[bidx.at[bl * _BODY_LEN + j]],
                                     bval.at[bl * _BODY_LEN + j], sgb)
                for j in range(_HEAD_LEN):
                    pltpu.async_copy(spmem.at[hidx.at[bl * _HEAD_LEN + j]],
                                     hval.at[bl * _HEAD_LEN + j], sgh)
                return 0

            lax.fori_loop(0, _CBL, fire, 0)

        def drain_i(i):
            bidx, bval, hidx, hval, _, _, sgb, sgh = bufs[i % 2]

            def drain(bl, _):
                for j in range(_BODY_LEN):
                    pltpu.make_async_copy(
                        spmem.at[bidx.at[bl * _BODY_LEN + j]],
                        bval.at[bl * _BODY_LEN + j], sgb).wait()
                for j in range(_HEAD_LEN):
                    pltpu.make_async_copy(
                        spmem.at[hidx.at[bl * _HEAD_LEN + j]],
                        hval.at[bl * _HEAD_LEN + j], sgh).wait()
                return 0

            lax.fori_loop(0, _CBL, drain, 0)

        def compute_i(i, acc):
            r, c = sched[i]
            _, bval, _, hval, _, _, _, _ = bufs[i % 2]
            q_base = base_q + c * (_CBL // 4)

            def block_body(bl, a):
                q = q_base + lax.shift_right_logical(bl, 2)
                keep = jnp.logical_and(q >= tq0, q < tq1)
                f = jnp.where(keep, jnp.float32(1.0), jnp.float32(0.0))
                for k in range(_BLK // _LANES):
                    s = k * _LANES
                    t = bval[bl * _BODY_LEN, pl.ds(s, _LANES)]
                    for j in range(1, _BODY_LEN):
                        t = t * bval[bl * _BODY_LEN + j, pl.ds(s, _LANES)]
                    for j in range(_HEAD_LEN):
                        t = t * (jnp.float32(1.0)
                                 - hval[bl * _HEAD_LEN + j,
                                        pl.ds(s, _LANES)])
                    a = a + t * f
                return a

            return lax.fori_loop(0, _CBL, block_body, acc)

        handles = stage_i(0)
        fire_i(0, handles)
        handles = stage_i(1)
        acc = jnp.zeros((_LANES,), jnp.float32)
        for i in range(nsched):
            drain_i(i)
            if i + 2 < nsched:
                next_handles = stage_i(i + 2)
            if i + 1 < nsched:
                fire_i(i + 1, handles)
                handles = next_handles if i + 2 < nsched else None
            acc = compute_i(i, acc)
            r, c = sched[i]
            if c == _NCH - 1:
                stage[...] = acc
                pltpu.sync_copy(stage, out_hbm.at[r, wid])
                acc = jnp.zeros((_LANES,), jnp.float32)

    return sc_kernel


_SC_KERNEL = _make_sc_kernel()


def _to_blocks(x, k):
    return (x[:_NBLK * _BLK].reshape(_NBLK, _BLK, k).transpose(0, 2, 1)
            .reshape(_NBLK * k, _BLK))


def kernel(predictions, rule_weights, A_in_0, A_out_0, A_in_1, A_out_1,
           A_in_2, A_out_2):
    ains = [_to_blocks(a, _BODY_LEN) for a in (A_in_0, A_in_1, A_in_2)]
    aouts = [_to_blocks(a, _HEAD_LEN) for a in (A_out_0, A_out_1, A_out_2)]
    partials = _SC_KERNEL(predictions, ains[0], aouts[0], ains[1], aouts[1],
                          ains[2], aouts[2])
    s = partials.sum(axis=(1, 2))

    tail = []
    for a_in, a_out in ((A_in_0, A_out_0), (A_in_1, A_out_1),
                        (A_in_2, A_out_2)):
        tb = jnp.prod(jnp.take(predictions, a_in[_NBLK * _BLK:], axis=0),
                      axis=-1)
        th = jnp.prod(1.0 - jnp.take(predictions, a_out[_NBLK * _BLK:],
                                     axis=0), axis=-1)
        tail.append(jnp.sum(tb * th))
    s = s + jnp.stack(tail)

    return jnp.sum(rule_weights * s) / jnp.float32(_N_GROUND)


# speedup vs baseline: 12.5468x; 1.0103x over previous
import functools

import jax
import jax.numpy as jnp
from jax import lax
from jax.experimental import pallas as pl
from jax.experimental.pallas import tpu as pltpu, tpu_sc as plsc

_N_ATOMS = 1000000
_N_GROUND = 500000
_BODY_LEN = 4
_HEAD_LEN = 2
_LANES = 16

_BLK = 128
_NBLK = 3904
_NQ = _NBLK // 4
_NW = 32
_CBL = 16
_NCH = 8
_NRULE = 3


def _make_sc_kernel():
    mesh = plsc.VectorSubcoreMesh(core_axis_name="c", subcore_axis_name="s")

    @functools.partial(
        pl.kernel,
        mesh=mesh,
        out_type=jax.ShapeDtypeStruct((_NRULE, _NW, _LANES), jnp.float32),
        compiler_params=pltpu.CompilerParams(
            needs_layout_passes=False,
            use_tc_tiling_on_sc=False,
        ),
        scratch_types=[
            pltpu.VMEM_SHARED((_N_ATOMS,), jnp.float32),
            pltpu.VMEM((_CBL * _BODY_LEN, _BLK), jnp.int32),
            pltpu.VMEM((_CBL * _BODY_LEN, _BLK), jnp.float32),
            pltpu.VMEM((_CBL * _HEAD_LEN, _BLK), jnp.int32),
            pltpu.VMEM((_CBL * _HEAD_LEN, _BLK), jnp.float32),
            pltpu.VMEM((_CBL * _BODY_LEN, _BLK), jnp.int32),
            pltpu.VMEM((_CBL * _BODY_LEN, _BLK), jnp.float32),
            pltpu.VMEM((_CBL * _HEAD_LEN, _BLK), jnp.int32),
            pltpu.VMEM((_CBL * _HEAD_LEN, _BLK), jnp.float32),
            pltpu.VMEM((_LANES,), jnp.float32),
            pltpu.SemaphoreType.DMA, pltpu.SemaphoreType.DMA,
            pltpu.SemaphoreType.DMA, pltpu.SemaphoreType.DMA,
            pltpu.SemaphoreType.DMA, pltpu.SemaphoreType.DMA,
            pltpu.SemaphoreType.DMA, pltpu.SemaphoreType.DMA,
        ],
    )
    def sc_kernel(pred_hbm, ain0, aout0, ain1, aout1, ain2, aout2,
                  dummy_b, dummy_h, out_hbm,
                  spmem, bidx0, bval0, hidx0, hval0,
                  bidx1, bval1, hidx1, hval1, stage,
                  ssb0, ssh0, ssb1, ssh1, sgb0, sgh0, sgb1, sgh1):
        cid = lax.axis_index("c")
        sid = lax.axis_index("s")
        wid = sid * 2 + cid

        @pl.when(sid == 0)
        def _stage_table():
            pltpu.sync_copy(pred_hbm, spmem)

        plsc.subcore_barrier()

        tq0 = jnp.where(wid < 16, 31 * wid, 496 + 30 * (wid - 16))
        nq = jnp.where(wid < 16, 31, 30)
        tq1 = tq0 + nq
        base_q = jnp.minimum(tq0, _NQ - _NCH * (_CBL // 4))

        rules = ((ain0, aout0), (ain1, aout1), (ain2, aout2))
        bufs = ((bidx0, bval0, hidx0, hval0, ssb0, ssh0, sgb0, sgh0),
                (bidx1, bval1, hidx1, hval1, ssb1, ssh1, sgb1, sgh1))
        sched = [(r, c) for r in range(_NRULE) for c in range(_NCH)]
        nsched = len(sched)

        def stage_i(i):
            r, c = sched[i]
            bidx, _, hidx, _, ssb, ssh, _, _ = bufs[i % 2]
            ain, aout = rules[r]
            q0 = base_q + c * (_CBL // 4)
            hs = pltpu.async_copy(
                ain.at[pl.ds(q0 * 4 * _BODY_LEN, _CBL * _BODY_LEN), :],
                bidx, ssb)
            hh = pltpu.async_copy(
                aout.at[pl.ds(q0 * 4 * _HEAD_LEN, _CBL * _HEAD_LEN), :],
                hidx, ssh)
            return hs, hh

        def fire_i(i, handles):
            hs, hh = handles
            hs.wait()
            hh.wait()
            bidx, bval, hidx, hval, _, _, sgb, sgh = bufs[i % 2]

            def fire(bl, _):
                for j in range(_BODY_LEN):
                    pltpu.async_copy(spmem.at[bidx.at[bl * _BODY_LEN + j]],
                                     bval.at[bl * _BODY_LEN + j], sgb)
                for j in range(_HEAD_LEN):
                    pltpu.async_copy(spmem.at[hidx.at[bl * _HEAD_LEN + j]],
                                     hval.at[bl * _HEAD_LEN + j], sgh)
                return 0

            lax.fori_loop(0, _CBL, fire, 0)

        def drain_i(i):
            _, bval, _, hval, _, _, sgb, sgh = bufs[i % 2]
            pltpu.make_async_copy(dummy_b, bval, sgb).wait()
            pltpu.make_async_copy(dummy_h, hval, sgh).wait()

        def compute_i(i, acc):
            r, c = sched[i]
            _, bval, _, hval, _, _, _, _ = bufs[i % 2]
            q_base = base_q + c * (_CBL // 4)

            def block_body(bl, a):
                q = q_base + lax.shift_right_logical(bl, 2)
                keep = jnp.logical_and(q >= tq0, q < tq1)
                f = jnp.where(keep, jnp.float32(1.0), jnp.float32(0.0))
                for k in range(_BLK // _LANES):
                    s = k * _LANES
                    t = bval[bl * _BODY_LEN, pl.ds(s, _LANES)]
                    for j in range(1, _BODY_LEN):
                        t = t * bval[bl * _BODY_LEN + j, pl.ds(s, _LANES)]
                    for j in range(_HEAD_LEN):
                        t = t * (jnp.float32(1.0)
                                 - hval[bl * _HEAD_LEN + j,
                                        pl.ds(s, _LANES)])
                    a = a + t * f
                return a

            return lax.fori_loop(0, _CBL, block_body, acc)

        handles = stage_i(0)
        fire_i(0, handles)
        handles = stage_i(1)
        acc = jnp.zeros((_LANES,), jnp.float32)
        for i in range(nsched):
            drain_i(i)
            if i + 2 < nsched:
                next_handles = stage_i(i + 2)
            if i + 1 < nsched:
                fire_i(i + 1, handles)
                handles = next_handles if i + 2 < nsched else None
            acc = compute_i(i, acc)
            r, c = sched[i]
            if c == _NCH - 1:
                stage[...] = acc
                pltpu.sync_copy(stage, out_hbm.at[r, wid])
                acc = jnp.zeros((_LANES,), jnp.float32)

    return sc_kernel


_SC_KERNEL = _make_sc_kernel()


def _to_blocks(x, k):
    return (x[:_NBLK * _BLK].reshape(_NBLK, _BLK, k).transpose(0, 2, 1)
            .reshape(_NBLK * k, _BLK))


def kernel(predictions, rule_weights, A_in_0, A_out_0, A_in_1, A_out_1,
           A_in_2, A_out_2):
    ains = [_to_blocks(a, _BODY_LEN) for a in (A_in_0, A_in_1, A_in_2)]
    aouts = [_to_blocks(a, _HEAD_LEN) for a in (A_out_0, A_out_1, A_out_2)]
    dummy_b = jnp.zeros((_CBL * _BODY_LEN, _BLK), jnp.float32)
    dummy_h = jnp.zeros((_CBL * _HEAD_LEN, _BLK), jnp.float32)
    partials = _SC_KERNEL(predictions, ains[0], aouts[0], ains[1], aouts[1],
                          ains[2], aouts[2], dummy_b, dummy_h)
    s = partials.sum(axis=(1, 2))

    tail = []
    for a_in, a_out in ((A_in_0, A_out_0), (A_in_1, A_out_1),
                        (A_in_2, A_out_2)):
        tb = jnp.prod(jnp.take(predictions, a_in[_NBLK * _BLK:], axis=0),
                      axis=-1)
        th = jnp.prod(1.0 - jnp.take(predictions, a_out[_NBLK * _BLK:],
                                     axis=0), axis=-1)
        tail.append(jnp.sum(tb * th))
    s = s + jnp.stack(tail)

    return jnp.sum(rule_weights * s) / jnp.float32(_N_GROUND)
